# baseline (speedup 1.0000x reference)
import functools

import jax
import jax.numpy as jnp
from jax import lax
from jax.experimental import pallas as pl
from jax.experimental.pallas import tpu as pltpu
from jax.experimental.pallas import tpu_sc as plsc


def _sc_factual_copy(st_t):
    n, d, b = st_t.shape
    info = plsc.get_sparse_core_info()
    nw = info.num_cores * info.num_subcores
    dc = 8
    chunks = n * (d // dc)
    per_w = chunks // nw

    mesh = plsc.VectorSubcoreMesh(core_axis_name="c", subcore_axis_name="s")

    @functools.partial(
        pl.kernel, mesh=mesh,
        out_type=jax.ShapeDtypeStruct((n, d, b), jnp.float32),
        scratch_types=[
            pltpu.VMEM((dc, b), jnp.float32),
            pltpu.VMEM((dc, b), jnp.float32),
            pltpu.SemaphoreType.DMA,
            pltpu.SemaphoreType.DMA,
        ],
    )
    def _copy(st_hbm, out_hbm, buf0, buf1, sem0, sem1):
        wid = lax.axis_index("s") * info.num_cores + lax.axis_index("c")
        base = wid * per_w
        bufs = (buf0, buf1)
        sems = (sem0, sem1)
        for u in range(per_w):
            c = base + u
            i = c // (d // dc)
            d0 = (c % (d // dc)) * dc
            buf, sem = bufs[u % 2], sems[u % 2]
            pltpu.async_copy(st_hbm.at[i, pl.ds(d0, dc), :], buf, sem).wait()
            pltpu.async_copy(buf, out_hbm.at[i, pl.ds(d0, dc), :], sem).wait()

    return _copy(st_t)


def _adj_kernel(tgt_ref, adj_ref, cf_adj_ref, edge_ref):
    bi, n, b = adj_ref.shape
    t = tgt_ref[...]
    tb = jnp.broadcast_to(t, (n, b))
    j_iota = jax.lax.broadcasted_iota(jnp.int32, (n, b), 0)
    keep = (j_iota != tb).astype(jnp.float32)

    adj = adj_ref[...]
    cf_adj_ref[...] = adj * keep[None, :, :]

    @pl.when(pl.program_id(0) == 0)
    def _():
        edge_ref[...] = jnp.zeros_like(edge_ref)

    i0 = pl.program_id(0) * bi
    acc = edge_ref[...]
    for li in range(bi):
        rowmask = (t == (i0 + li)).astype(jnp.float32)
        acc = acc + adj[li] * rowmask
    edge_ref[...] = acc


def _mlp_kernel(tgt_ref, st_ref, edge_ref, iv_ref,
                w1srcT_ref, w1stT_ref, w1w_ref, b1_ref, w2T_ref, b2_ref,
                cf_pred_ref):
    bn, d, b = st_ref.shape
    t = tgt_ref[...]
    iv = iv_ref[...]
    ivp = (jnp.dot(w1srcT_ref[...], iv,
                   preferred_element_type=jnp.float32) + b1_ref[...])
    n0 = pl.program_id(0) * bn
    for ln in range(bn):
        stn = st_ref[ln]
        ew = edge_ref[ln:ln + 1, :]
        pre = (jnp.dot(w1stT_ref[...], stn,
                       preferred_element_type=jnp.float32)
               + ivp + w1w_ref[...] * ew)
        h = jnp.maximum(pre, 0.0)
        slot = jnp.tanh(jnp.dot(w2T_ref[...], h,
                                preferred_element_type=jnp.float32)
                        + b2_ref[...])
        m = (t == (n0 + ln)).astype(jnp.float32)
        cf_pred_ref[ln] = stn + m * (iv - stn) + slot * ew


@functools.partial(jax.jit, static_argnames=("interpret",))
def kernel(states, adjacency, target_idx, intervention_value,
           W1, b1, W2, b2, interpret=False):
    B, N, D = states.shape
    BI = 8
    BN = 8

    st_t = jnp.transpose(states, (1, 2, 0))
    adj_t = jnp.transpose(adjacency, (1, 2, 0))
    iv_t = intervention_value.T
    tgt2 = target_idx.astype(jnp.int32).reshape(1, B)
    W1T = W1.T
    w1srcT = W1T[:, :D]
    w1stT = W1T[:, D:2 * D]
    w1w = W1T[:, 2 * D:2 * D + 1]
    b1c = b1.reshape(D, 1)
    b2c = b2.reshape(D, 1)
    W2T = W2.T

    cf_adj_t, edge_w = pl.pallas_call(
        _adj_kernel,
        grid=(N // BI,),
        in_specs=[
            pl.BlockSpec((1, B), lambda i: (0, 0)),
            pl.BlockSpec((BI, N, B), lambda i: (i, 0, 0)),
        ],
        out_specs=[
            pl.BlockSpec((BI, N, B), lambda i: (i, 0, 0)),
            pl.BlockSpec((N, B), lambda i: (0, 0)),
        ],
        out_shape=[
            jax.ShapeDtypeStruct((N, N, B), jnp.float32),
            jax.ShapeDtypeStruct((N, B), jnp.float32),
        ],
        compiler_params=pltpu.CompilerParams(
            dimension_semantics=("arbitrary",),
        ),
        interpret=interpret,
    )(tgt2, adj_t)

    if interpret:
        fact_t = st_t + 0.0
    else:
        fact_t = _sc_factual_copy(st_t)

    cf_pred_t = pl.pallas_call(
        _mlp_kernel,
        grid=(N // BN,),
        in_specs=[
            pl.BlockSpec((1, B), lambda i: (0, 0)),
            pl.BlockSpec((BN, D, B), lambda i: (i, 0, 0)),
            pl.BlockSpec((BN, B), lambda i: (i, 0)),
            pl.BlockSpec((D, B), lambda i: (0, 0)),
            pl.BlockSpec((D, D), lambda i: (0, 0)),
            pl.BlockSpec((D, D), lambda i: (0, 0)),
            pl.BlockSpec((D, 1), lambda i: (0, 0)),
            pl.BlockSpec((D, 1), lambda i: (0, 0)),
            pl.BlockSpec((D, D), lambda i: (0, 0)),
            pl.BlockSpec((D, 1), lambda i: (0, 0)),
        ],
        out_specs=pl.BlockSpec((BN, D, B), lambda i: (i, 0, 0)),
        out_shape=jax.ShapeDtypeStruct((N, D, B), jnp.float32),
        compiler_params=pltpu.CompilerParams(
            dimension_semantics=("arbitrary",),
        ),
        interpret=interpret,
    )(tgt2, st_t, edge_w, iv_t, w1srcT, w1stT, w1w, b1c, W2T, b2c)

    cf_pred = jnp.transpose(cf_pred_t, (2, 0, 1)).reshape(B, 1, N, D)
    fact = jnp.transpose(fact_t, (2, 0, 1)).reshape(B, 1, N, D)
    cf_adj = jnp.transpose(cf_adj_t, (2, 0, 1))
    return (cf_pred, fact, cf_adj, target_idx, intervention_value)


# speedup vs baseline: 1.3560x; 1.3560x over previous
import functools

import jax
import jax.numpy as jnp
from jax.experimental import pallas as pl
from jax.experimental.pallas import tpu as pltpu


def _adj_kernel(tgt_ref, adj_ref, cf_adj_ref, edge_ref):
    bi, n, b = adj_ref.shape
    t = tgt_ref[...]
    tb = jnp.broadcast_to(t, (n, b))
    j_iota = jax.lax.broadcasted_iota(jnp.int32, (n, b), 0)
    keep = (j_iota != tb).astype(jnp.float32)

    adj = adj_ref[...]
    cf_adj_ref[...] = adj * keep[None, :, :]

    @pl.when(pl.program_id(0) == 0)
    def _():
        edge_ref[...] = jnp.zeros_like(edge_ref)

    i0 = pl.program_id(0) * bi
    acc = edge_ref[...]
    for li in range(bi):
        rowmask = (t == (i0 + li)).astype(jnp.float32)
        acc = acc + adj[li] * rowmask
    edge_ref[...] = acc


def _mlp_kernel(tgt_ref, st_ref, edge_ref, iv_ref,
                w1srcT_ref, w1stT_ref, w1w_ref, b1_ref, w2T_ref, b2_ref,
                cf_pred_ref, fact_ref):
    bn, d, b = st_ref.shape
    t = tgt_ref[...]
    iv = iv_ref[...]
    ivp = (jnp.dot(w1srcT_ref[...], iv,
                   preferred_element_type=jnp.float32) + b1_ref[...])
    n0 = pl.program_id(0) * bn
    for ln in range(bn):
        stn = st_ref[ln]
        ew = edge_ref[ln:ln + 1, :]
        pre = (jnp.dot(w1stT_ref[...], stn,
                       preferred_element_type=jnp.float32)
               + ivp + w1w_ref[...] * ew)
        h = jnp.maximum(pre, 0.0)
        slot = jnp.tanh(jnp.dot(w2T_ref[...], h,
                                preferred_element_type=jnp.float32)
                        + b2_ref[...])
        m = (t == (n0 + ln)).astype(jnp.float32)
        cf_pred_ref[ln] = stn + m * (iv - stn) + slot * ew
        fact_ref[ln] = stn


@functools.partial(jax.jit, static_argnames=("interpret",))
def kernel(states, adjacency, target_idx, intervention_value,
           W1, b1, W2, b2, interpret=False):
    B, N, D = states.shape
    BI = 8
    BN = 16

    st_t = jnp.transpose(states, (1, 2, 0))
    adj_t = jnp.transpose(adjacency, (1, 2, 0))
    iv_t = intervention_value.T
    tgt2 = target_idx.astype(jnp.int32).reshape(1, B)
    W1T = W1.T
    w1srcT = W1T[:, :D]
    w1stT = W1T[:, D:2 * D]
    w1w = W1T[:, 2 * D:2 * D + 1]
    b1c = b1.reshape(D, 1)
    b2c = b2.reshape(D, 1)
    W2T = W2.T

    cf_adj_t, edge_w = pl.pallas_call(
        _adj_kernel,
        grid=(N // BI,),
        in_specs=[
            pl.BlockSpec((1, B), lambda i: (0, 0)),
            pl.BlockSpec((BI, N, B), lambda i: (i, 0, 0)),
        ],
        out_specs=[
            pl.BlockSpec((BI, N, B), lambda i: (i, 0, 0)),
            pl.BlockSpec((N, B), lambda i: (0, 0)),
        ],
        out_shape=[
            jax.ShapeDtypeStruct((N, N, B), jnp.float32),
            jax.ShapeDtypeStruct((N, B), jnp.float32),
        ],
        compiler_params=pltpu.CompilerParams(
            dimension_semantics=("arbitrary",),
        ),
        interpret=interpret,
    )(tgt2, adj_t)

    cf_pred_t, fact_t = pl.pallas_call(
        _mlp_kernel,
        grid=(N // BN,),
        in_specs=[
            pl.BlockSpec((1, B), lambda i: (0, 0)),
            pl.BlockSpec((BN, D, B), lambda i: (i, 0, 0)),
            pl.BlockSpec((BN, B), lambda i: (i, 0)),
            pl.BlockSpec((D, B), lambda i: (0, 0)),
            pl.BlockSpec((D, D), lambda i: (0, 0)),
            pl.BlockSpec((D, D), lambda i: (0, 0)),
            pl.BlockSpec((D, 1), lambda i: (0, 0)),
            pl.BlockSpec((D, 1), lambda i: (0, 0)),
            pl.BlockSpec((D, D), lambda i: (0, 0)),
            pl.BlockSpec((D, 1), lambda i: (0, 0)),
        ],
        out_specs=[
            pl.BlockSpec((BN, D, B), lambda i: (i, 0, 0)),
            pl.BlockSpec((BN, D, B), lambda i: (i, 0, 0)),
        ],
        out_shape=[
            jax.ShapeDtypeStruct((N, D, B), jnp.float32),
            jax.ShapeDtypeStruct((N, D, B), jnp.float32),
        ],
        compiler_params=pltpu.CompilerParams(
            dimension_semantics=("arbitrary",),
        ),
        interpret=interpret,
    )(tgt2, st_t, edge_w, iv_t, w1srcT, w1stT, w1w, b1c, W2T, b2c)

    cf_pred = jnp.transpose(cf_pred_t, (2, 0, 1)).reshape(B, 1, N, D)
    fact = jnp.transpose(fact_t, (2, 0, 1)).reshape(B, 1, N, D)
    cf_adj = jnp.transpose(cf_adj_t, (2, 0, 1))
    return (cf_pred, fact, cf_adj, target_idx, intervention_value)


# speedup vs baseline: 1.3694x; 1.0099x over previous
import jax
import jax.numpy as jnp
from jax.experimental import pallas as pl
from jax.experimental.pallas import tpu as pltpu


def _adj_kernel(tgt_ref, adj_ref, cf_adj_ref, edge_ref):
    bi, n, b = adj_ref.shape
    t = tgt_ref[...]
    tb = jnp.broadcast_to(t, (n, b))
    j_iota = jax.lax.broadcasted_iota(jnp.int32, (n, b), 0)
    keep = (j_iota != tb).astype(jnp.float32)

    adj = adj_ref[...]
    cf_adj_ref[...] = adj * keep[None, :, :]

    @pl.when(pl.program_id(0) == 0)
    def _():
        edge_ref[...] = jnp.zeros_like(edge_ref)

    i0 = pl.program_id(0) * bi
    acc = edge_ref[...]
    for li in range(bi):
        rowmask = (t == (i0 + li)).astype(jnp.float32)
        acc = acc + adj[li] * rowmask
    edge_ref[...] = acc


def _mlp_kernel(tgt_ref, st_ref, edge_ref, iv_ref,
                w1srcT_ref, w1stT_ref, w1w_ref, b1_ref, w2T_ref, b2_ref,
                cf_pred_ref, fact_ref):
    bn, d, b = st_ref.shape
    t = tgt_ref[...]
    iv = iv_ref[...]
    ivp = (jnp.dot(w1srcT_ref[...], iv,
                   preferred_element_type=jnp.float32) + b1_ref[...])
    n0 = pl.program_id(0) * bn
    for ln in range(bn):
        stn = st_ref[ln]
        ew = edge_ref[ln:ln + 1, :]
        pre = (jnp.dot(w1stT_ref[...], stn,
                       preferred_element_type=jnp.float32)
               + ivp + w1w_ref[...] * ew)
        h = jnp.maximum(pre, 0.0)
        slot = jnp.tanh(jnp.dot(w2T_ref[...], h,
                                preferred_element_type=jnp.float32)
                        + b2_ref[...])
        m = (t == (n0 + ln)).astype(jnp.float32)
        cf_pred_ref[ln] = stn + m * (iv - stn) + slot * ew
        fact_ref[ln] = stn


def kernel(states, adjacency, target_idx, intervention_value,
           W1, b1, W2, b2):
    B, N, D = states.shape
    BI = 8
    BN = 8

    st_t = jnp.transpose(states, (1, 2, 0))
    adj_t = jnp.transpose(adjacency, (1, 2, 0))
    iv_t = intervention_value.T
    tgt2 = target_idx.astype(jnp.int32).reshape(1, B)
    W1T = W1.T
    w1srcT = W1T[:, :D]
    w1stT = W1T[:, D:2 * D]
    w1w = W1T[:, 2 * D:2 * D + 1]
    b1c = b1.reshape(D, 1)
    b2c = b2.reshape(D, 1)
    W2T = W2.T

    cf_adj_t, edge_w = pl.pallas_call(
        _adj_kernel,
        grid=(N // BI,),
        in_specs=[
            pl.BlockSpec((1, B), lambda i: (0, 0)),
            pl.BlockSpec((BI, N, B), lambda i: (i, 0, 0)),
        ],
        out_specs=[
            pl.BlockSpec((BI, N, B), lambda i: (i, 0, 0)),
            pl.BlockSpec((N, B), lambda i: (0, 0)),
        ],
        out_shape=[
            jax.ShapeDtypeStruct((N, N, B), jnp.float32),
            jax.ShapeDtypeStruct((N, B), jnp.float32),
        ],
        compiler_params=pltpu.CompilerParams(
            dimension_semantics=("arbitrary",),
        ),
    )(tgt2, adj_t)

    cf_pred_t, fact_t = pl.pallas_call(
        _mlp_kernel,
        grid=(N // BN,),
        in_specs=[
            pl.BlockSpec((1, B), lambda i: (0, 0)),
            pl.BlockSpec((BN, D, B), lambda i: (i, 0, 0)),
            pl.BlockSpec((BN, B), lambda i: (i, 0)),
            pl.BlockSpec((D, B), lambda i: (0, 0)),
            pl.BlockSpec((D, D), lambda i: (0, 0)),
            pl.BlockSpec((D, D), lambda i: (0, 0)),
            pl.BlockSpec((D, 1), lambda i: (0, 0)),
            pl.BlockSpec((D, 1), lambda i: (0, 0)),
            pl.BlockSpec((D, D), lambda i: (0, 0)),
            pl.BlockSpec((D, 1), lambda i: (0, 0)),
        ],
        out_specs=[
            pl.BlockSpec((BN, D, B), lambda i: (i, 0, 0)),
            pl.BlockSpec((BN, D, B), lambda i: (i, 0, 0)),
        ],
        out_shape=[
            jax.ShapeDtypeStruct((N, D, B), jnp.float32),
            jax.ShapeDtypeStruct((N, D, B), jnp.float32),
        ],
        compiler_params=pltpu.CompilerParams(
            dimension_semantics=("arbitrary",),
        ),
    )(tgt2, st_t, edge_w, iv_t, w1srcT, w1stT, w1w, b1c, W2T, b2c)

    cf_pred = jnp.transpose(cf_pred_t, (2, 0, 1)).reshape(B, 1, N, D)
    fact = jnp.transpose(fact_t, (2, 0, 1)).reshape(B, 1, N, D)
    cf_adj = jnp.transpose(cf_adj_t, (2, 0, 1))
    return (cf_pred, fact, cf_adj, target_idx, intervention_value)
